# baseline (speedup 1.0000x reference)
import functools

import jax
import jax.numpy as jnp
from jax import lax
from jax.experimental import pallas as pl
from jax.experimental.pallas import tpu as pltpu
from jax.experimental.pallas import tpu_sc as plsc

DIM = 768
LANES = 16
KCHUNKS = DIM // LANES
EPS = 1e-12

NC = 2
NS = 16
NW = NC * NS


def _lane_sum_splat(v):
    idx = lax.iota(jnp.int32, LANES)
    for sh in (8, 4, 2, 1):
        perm = jnp.bitwise_xor(idx, sh)
        v = v + v.at[perm].get(mode="promise_in_bounds")
    return v


def _rsqrt_vec(x):
    i = lax.bitcast_convert_type(x, jnp.int32)
    i = jnp.int32(0x5F3759DF) - lax.shift_right_logical(i, 1)
    y = lax.bitcast_convert_type(i, jnp.float32)
    for _ in range(3):
        y = y * (1.5 - 0.5 * x * y * y)
    return y


def _make_sc_call(total_rows, seq_len, chunk):
    rows_per_w = total_rows // NW
    n_chunks = rows_per_w // chunk
    mesh = plsc.VectorSubcoreMesh(
        core_axis_name="c", subcore_axis_name="s",
        num_cores=NC, num_subcores=NS)

    @functools.partial(
        pl.kernel,
        out_type=jax.ShapeDtypeStruct((total_rows, DIM), jnp.float32),
        mesh=mesh,
        scratch_types=[
            pltpu.VMEM((chunk,), jnp.int32),
            pltpu.VMEM((chunk, DIM), jnp.float32),
            pltpu.VMEM((chunk, DIM), jnp.float32),
            pltpu.VMEM((DIM,), jnp.float32),
            pltpu.VMEM((DIM,), jnp.float32),
            pltpu.SemaphoreType.DMA,
        ],
    )
    def sc_embed(ids_hbm, tok_hbm, pos_hbm, gamma_hbm, beta_hbm, out_hbm,
                 idx_v, xb, pb, gamma_v, beta_v, sem):
        wid = lax.axis_index("s") * NC + lax.axis_index("c")
        base = wid * rows_per_w

        pltpu.sync_copy(gamma_hbm, gamma_v)
        pltpu.sync_copy(beta_hbm, beta_v)

        def chunk_body(ci, _):
            off = base + ci * chunk
            poff = lax.rem(off, seq_len)
            pltpu.sync_copy(ids_hbm.at[pl.ds(off, chunk)], idx_v)
            pltpu.async_copy(tok_hbm.at[idx_v], xb, sem).wait()
            pltpu.sync_copy(pos_hbm.at[pl.ds(poff, chunk), :], pb)

            def token_body(j, _):
                zero = jnp.zeros((LANES,), jnp.float32)

                def pass1(k, carry):
                    s, q = carry
                    sl = pl.ds(k * LANES, LANES)
                    v = xb[j, sl] + pb[j, sl]
                    xb[j, sl] = v
                    return s + v, q + v * v

                s, q = lax.fori_loop(0, KCHUNKS, pass1, (zero, zero))
                s_vec = _lane_sum_splat(s) * (1.0 / DIM)
                q_vec = _lane_sum_splat(q) * (1.0 / DIM)
                var_vec = q_vec - s_vec * s_vec
                r_vec = _rsqrt_vec(var_vec + EPS)

                def pass2(k, _):
                    sl = pl.ds(k * LANES, LANES)
                    x = xb[j, sl]
                    xb[j, sl] = (x - s_vec) * r_vec * gamma_v[sl] + beta_v[sl]
                    return 0

                lax.fori_loop(0, KCHUNKS, pass2, 0)
                return 0

            lax.fori_loop(0, chunk, token_body, 0)
            pltpu.sync_copy(xb, out_hbm.at[pl.ds(off, chunk), :])
            return 0

        lax.fori_loop(0, n_chunks, chunk_body, 0)

    return sc_embed


def kernel(input_ids, token_table, pos_table, ln_gamma, ln_beta):
    batch, seq_len = input_ids.shape
    total_rows = batch * seq_len
    ids_flat = input_ids.reshape(total_rows).astype(jnp.int32)
    sc_call = _make_sc_call(total_rows, seq_len, chunk=64)
    out = sc_call(ids_flat, token_table, pos_table, ln_gamma, ln_beta)
    return out.reshape(batch, seq_len, DIM)


# speedup vs baseline: 1.6148x; 1.6148x over previous
import functools

import jax
import jax.numpy as jnp
from jax import lax
from jax.experimental import pallas as pl
from jax.experimental.pallas import tpu as pltpu
from jax.experimental.pallas import tpu_sc as plsc

DIM = 768
LANES = 16
KCHUNKS = DIM // LANES
EPS = 1e-12

NC = 2
NS = 16
NW = NC * NS


def _lane_sum_splat(v):
    idx = lax.iota(jnp.int32, LANES)
    for sh in (8, 4, 2, 1):
        perm = jnp.bitwise_xor(idx, sh)
        v = v + v.at[perm].get(mode="promise_in_bounds")
    return v


def _rsqrt_vec(x):
    i = lax.bitcast_convert_type(x, jnp.int32)
    i = jnp.int32(0x5F3759DF) - lax.shift_right_logical(i, 1)
    y = lax.bitcast_convert_type(i, jnp.float32)
    for _ in range(3):
        y = y * (1.5 - 0.5 * x * y * y)
    return y


def _make_sc_call(total_rows, seq_len, chunk):
    rows_per_w = total_rows // NW
    n_chunks = rows_per_w // chunk
    mesh = plsc.VectorSubcoreMesh(
        core_axis_name="c", subcore_axis_name="s",
        num_cores=NC, num_subcores=NS)

    @functools.partial(
        pl.kernel,
        out_type=jax.ShapeDtypeStruct((total_rows, DIM), jnp.float32),
        mesh=mesh,
        scratch_types=[
            pltpu.VMEM((chunk,), jnp.int32),
            pltpu.VMEM((chunk, DIM), jnp.float32),
            pltpu.VMEM((chunk, DIM), jnp.float32),
            pltpu.VMEM((DIM,), jnp.float32),
            pltpu.VMEM((DIM,), jnp.float32),
            pltpu.SemaphoreType.DMA,
        ],
    )
    def sc_embed(ids_hbm, tok_hbm, pos_hbm, gamma_hbm, beta_hbm, out_hbm,
                 idx_v, xb, pb, gamma_v, beta_v, sem):
        wid = lax.axis_index("s") * NC + lax.axis_index("c")
        base = wid * rows_per_w

        pltpu.sync_copy(gamma_hbm, gamma_v)
        pltpu.sync_copy(beta_hbm, beta_v)

        def chunk_body(ci, _):
            off = base + ci * chunk
            poff = lax.rem(off, seq_len)
            pltpu.sync_copy(ids_hbm.at[pl.ds(off, chunk)], idx_v)
            pltpu.async_copy(tok_hbm.at[idx_v], xb, sem).wait()
            pltpu.sync_copy(pos_hbm.at[pl.ds(poff, chunk), :], pb)

            def token_body(j, _):
                zero = jnp.zeros((LANES,), jnp.float32)
                s, q = zero, zero
                for k in range(KCHUNKS):
                    sl = pl.ds(k * LANES, LANES)
                    v = xb[j, sl] + pb[j, sl]
                    xb[j, sl] = v
                    s = s + v
                    q = q + v * v

                s_vec = _lane_sum_splat(s) * (1.0 / DIM)
                q_vec = _lane_sum_splat(q) * (1.0 / DIM)
                var_vec = q_vec - s_vec * s_vec
                r_vec = _rsqrt_vec(var_vec + EPS)

                for k in range(KCHUNKS):
                    sl = pl.ds(k * LANES, LANES)
                    x = xb[j, sl]
                    xb[j, sl] = (x - s_vec) * r_vec * gamma_v[sl] + beta_v[sl]
                return 0

            lax.fori_loop(0, chunk, token_body, 0)
            pltpu.sync_copy(xb, out_hbm.at[pl.ds(off, chunk), :])
            return 0

        lax.fori_loop(0, n_chunks, chunk_body, 0)

    return sc_embed


def kernel(input_ids, token_table, pos_table, ln_gamma, ln_beta):
    batch, seq_len = input_ids.shape
    total_rows = batch * seq_len
    ids_flat = input_ids.reshape(total_rows).astype(jnp.int32)
    sc_call = _make_sc_call(total_rows, seq_len, chunk=64)
    out = sc_call(ids_flat, token_table, pos_table, ln_gamma, ln_beta)
    return out.reshape(batch, seq_len, DIM)
